# baseline (speedup 1.0000x reference)
import functools

import jax
import jax.numpy as jnp
from jax import lax
from jax.experimental import pallas as pl
from jax.experimental.pallas import tpu as pltpu
from jax.experimental.pallas import tpu_sc as plsc

N = 10000
E = 320000
F = 128
STEPS = 8
TILES = 16
CHUNK = 128
CPT = 160
EPT = CPT * CHUNK
E_PAD = TILES * EPT
NP = 10752
PIECES = 6
PN = NP // PIECES
PNP = PN + 8
RPT = PN // TILES
SD = 1024


def _sc_agg_body(h_hbm, src_hbm, dst_hbm, adj_hbm, out_hbm,
                 packed, w1, wix, stage_d, gbuf0, gbuf1, sbuf0, sbuf1,
                 ixs0, ixs1, ixs2, ixs3, dsts0, agg_sh,
                 gsem0, gsem1, gsem2, gsem3):
    core = lax.axis_index("c")
    tid = lax.axis_index("s")
    lane = lax.iota(jnp.int32, 16)
    KG = CHUNK // 16

    def build_ix(ch, g, ixs):
        base = g * NP
        for k in range(KG):
            sl = pl.ds(k * 16, 16)
            ixs[sl] = lax.shift_right_logical(
                packed[pl.ds(ch * CHUNK + k * 16, 16)], 14) + base

    def gstart(buf, sem, ixs):
        pltpu.make_async_copy(h_hbm.at[ixs], buf, sem).start()

    def gwait(buf, sem):
        pltpu.make_async_copy(h_hbm.at[ixs0], buf, sem).wait()

    def process(c, nch, g, gbuf, gsem, ixs):
        gwait(gbuf, gsem)

        def scale_body(q, carry):
            wv16 = w1[pl.ds(c * CHUNK + q * 16, 16)]
            for l in range(16):
                wb = jnp.full((16,), wv16[l], jnp.float32)
                for f in range(F // 16):
                    sl = pl.ds(f * 16, 16)
                    e = q * 16 + l
                    gbuf[e, sl] = gbuf[e, sl] * wb
            return carry
        lax.fori_loop(0, KG, scale_body, 0)

        for k in range(KG):
            sl = pl.ds(k * 16, 16)
            dsts0[0, sl] = packed[pl.ds(c * CHUNK + k * 16, 16)] & 16383
        pltpu.sync_copy(gbuf, agg_sh.at[dsts0.at[0]], add=True)

        nc = c + 4

        @pl.when(nc < nch)
        def _():
            build_ix(nc, g, ixs)
            gstart(gbuf, gsem, ixs)

    def piece_body(qi, pcarry):
        q = core * (PIECES // 2) + qi
        qbase = q * PN

        pltpu.sync_copy(src_hbm.at[tid], packed)

        def pack_blk(blk, carry):
            pltpu.sync_copy(dst_hbm.at[tid].at[pl.ds(blk * SD, SD)], stage_d)

            def pk(k, kc):
                slp = pl.ds(blk * SD + k * 16, 16)
                packed[slp] = packed[slp] * 16384 + stage_d[pl.ds(k * 16, 16)]
                return kc
            lax.fori_loop(0, SD // 16, pk, 0)
            return carry
        lax.fori_loop(0, EPT // SD, pack_blk, 0)

        def wg_body(rr, carry):
            base = rr * 8
            for k in range(8):
                ch = base + k
                for k2 in range(KG):
                    sl = pl.ds(k2 * 16, 16)
                    p = packed[pl.ds(ch * CHUNK + k2 * 16, 16)]
                    wix[k, sl] = (lax.shift_right_logical(p, 14) * N
                                  + (p & 16383))
                pltpu.make_async_copy(
                    adj_hbm.at[wix.at[k]],
                    w1.at[pl.ds(ch * CHUNK, CHUNK)], gsem0).start()
            for k in range(8):
                ch = base + k
                pltpu.make_async_copy(
                    adj_hbm.at[wix.at[k]],
                    w1.at[pl.ds(ch * CHUNK, CHUNK)], gsem0).wait()
            return carry
        lax.fori_loop(0, CPT // 8, wg_body, 0)

        def gat16(v, idx):
            return lax.gather(
                v, idx[:, None],
                lax.GatherDimensionNumbers(
                    offset_dims=(), collapsed_slice_dims=(0,),
                    start_index_map=(0,)),
                slice_sizes=(1,),
                mode=lax.GatherScatterMode.PROMISE_IN_BOUNDS)

        def cpt_body(gr, cnt):
            off = gr * 16
            pv = packed[pl.ds(off, 16)]
            wv = w1[pl.ds(off, 16)]
            dl = (pv & 16383) - qbase
            m = (dl >= 0) & (dl < PN)
            mi = jnp.where(m, 1, 0)
            s = mi
            for d in (1, 2, 4, 8):
                sh = gat16(s, jnp.maximum(lane - d, 0))
                s = s + jnp.where(lane >= d, sh, 0)
            nk = s[15]
            ex = s - mi
            r = jnp.where(m, ex, nk + (lane - ex))
            perm = lane
            for l in range(16):
                perm = jnp.where(lane == r[l], l, perm)
            packed[pl.ds(cnt, 16)] = gat16(pv, perm) - qbase
            w1[pl.ds(cnt, 16)] = gat16(wv, perm)
            return cnt + nk
        cnt = lax.fori_loop(0, EPT // 16, cpt_body, jnp.int32(0))

        c2 = 4 * CHUNK
        pcnt = jnp.maximum((cnt + c2 - 1) // c2 * c2, c2)
        g0 = cnt // 16
        rem = cnt - g0 * 16
        pad_p = PN + (lane & 7)

        @pl.when(cnt < pcnt)
        def _():
            off0 = g0 * 16
            keep = lane < rem
            pv = packed[pl.ds(off0, 16)]
            wv = w1[pl.ds(off0, 16)]
            packed[pl.ds(off0, 16)] = jnp.where(keep, pv, pad_p)
            w1[pl.ds(off0, 16)] = jnp.where(keep, wv, 0.0)

        def pad_body(gg, carry):
            off = gg * 16
            packed[pl.ds(off, 16)] = pad_p
            w1[pl.ds(off, 16)] = jnp.zeros((16,), jnp.float32)
            return carry
        lax.fori_loop(g0 + 1, pcnt // 16, pad_body, 0)

        nch = pcnt // CHUNK

        def step_body(g, carry):
            def zero_body(r, zc):
                for k in range(F // 16):
                    gbuf0[r, pl.ds(k * 16, 16)] = jnp.zeros(
                        (16,), jnp.float32)
                return zc
            lax.fori_loop(0, CHUNK, zero_body, 0)
            for zj in range(RPT // CHUNK):
                pltpu.sync_copy(
                    gbuf0, agg_sh.at[pl.ds(tid * RPT + zj * CHUNK, CHUNK)])
            if RPT % CHUNK:
                pltpu.sync_copy(
                    gbuf0.at[pl.ds(0, RPT % CHUNK)],
                    agg_sh.at[pl.ds(tid * RPT + (RPT // CHUNK) * CHUNK,
                                    RPT % CHUNK)])

            build_ix(0, g, ixs0)
            gstart(gbuf0, gsem0, ixs0)
            build_ix(1, g, ixs1)
            gstart(gbuf1, gsem1, ixs1)
            build_ix(2, g, ixs2)
            gstart(sbuf0, gsem2, ixs2)
            build_ix(3, g, ixs3)
            gstart(sbuf1, gsem3, ixs3)

            plsc.subcore_barrier()

            def edge_body(i, ec):
                process(i * 4, nch, g, gbuf0, gsem0, ixs0)
                process(i * 4 + 1, nch, g, gbuf1, gsem1, ixs1)
                process(i * 4 + 2, nch, g, sbuf0, gsem2, ixs2)
                process(i * 4 + 3, nch, g, sbuf1, gsem3, ixs3)
                return ec
            lax.fori_loop(0, nch // 4, edge_body, 0)

            plsc.subcore_barrier()

            pltpu.sync_copy(
                agg_sh.at[pl.ds(tid * RPT, RPT)],
                out_hbm.at[g].at[pl.ds(qbase + tid * RPT, RPT)])
            return carry
        lax.fori_loop(0, STEPS, step_body, 0)
        return pcarry
    lax.fori_loop(0, PIECES // 2, piece_body, 0)


def _make_sc_agg():
    return pl.kernel(
        _sc_agg_body,
        out_type=jax.ShapeDtypeStruct((STEPS, NP, F), jnp.float32),
        mesh=plsc.VectorSubcoreMesh(
            core_axis_name="c", subcore_axis_name="s"),
        scratch_types=[
            pltpu.VMEM((EPT,), jnp.int32),
            pltpu.VMEM((EPT,), jnp.float32),
            pltpu.VMEM((8, CHUNK), jnp.int32),
            pltpu.VMEM((SD,), jnp.int32),
            pltpu.VMEM((CHUNK, F), jnp.float32),
            pltpu.VMEM((CHUNK, F), jnp.float32),
            pltpu.VMEM((CHUNK, F), jnp.float32),
            pltpu.VMEM((CHUNK, F), jnp.float32),
            pltpu.VMEM((CHUNK,), jnp.int32),
            pltpu.VMEM((CHUNK,), jnp.int32),
            pltpu.VMEM((CHUNK,), jnp.int32),
            pltpu.VMEM((CHUNK,), jnp.int32),
            pltpu.VMEM((1, CHUNK), jnp.int32),
            pltpu.VMEM_SHARED((PNP, F), jnp.float32),
            pltpu.SemaphoreType.DMA,
            pltpu.SemaphoreType.DMA,
            pltpu.SemaphoreType.DMA,
            pltpu.SemaphoreType.DMA,
        ],
    )


_sc_agg = _make_sc_agg()


def _mm_relu_body(a_ref, w_ref, b_ref, o_ref):
    acc = lax.dot_general(a_ref[...], w_ref[...], (((1,), (0,)), ((), ())),
                          preferred_element_type=jnp.float32)
    o_ref[...] = jnp.maximum(acc + b_ref[...], 0.0)


def _tc_mm_relu(a, w, b):
    m = a.shape[0]
    bm = 2048
    return pl.pallas_call(
        _mm_relu_body,
        grid=(m // bm,),
        in_specs=[
            pl.BlockSpec((bm, F), lambda i: (i, 0)),
            pl.BlockSpec((F, F), lambda i: (0, 0)),
            pl.BlockSpec((1, F), lambda i: (0, 0)),
        ],
        out_specs=pl.BlockSpec((bm, F), lambda i: (i, 0)),
        out_shape=jax.ShapeDtypeStruct((m, F), jnp.float32),
    )(a, w, b.reshape(1, F))


def _mm_relu_colsum_body(a_ref, w_ref, b_ref, o_ref):
    j = pl.program_id(1)
    acc = lax.dot_general(a_ref[0], w_ref[...], (((1,), (0,)), ((), ())),
                          preferred_element_type=jnp.float32)
    h = jnp.maximum(acc + b_ref[...], 0.0)
    part = jnp.sum(h, axis=0, keepdims=True)[None]

    @pl.when(j == 0)
    def _():
        o_ref[...] = part

    @pl.when(j > 0)
    def _():
        o_ref[...] = o_ref[...] + part


def _tc_mm_relu_colsum(a, w, b):
    bm = 2000
    bps = N // bm
    return pl.pallas_call(
        _mm_relu_colsum_body,
        grid=(STEPS, bps),
        in_specs=[
            pl.BlockSpec((1, bm, F), lambda s, j: (s, j, 0)),
            pl.BlockSpec((F, F), lambda s, j: (0, 0)),
            pl.BlockSpec((1, F), lambda s, j: (0, 0)),
        ],
        out_specs=pl.BlockSpec((1, 1, F), lambda s, j: (s, 0, 0)),
        out_shape=jax.ShapeDtypeStruct((STEPS, 1, F), jnp.float32),
    )(a, w, b.reshape(1, F)).reshape(STEPS, F)


def _head_body(sums_ref, wl1_ref, bl1_ref, wl2_ref, bl2_ref,
               wd1_ref, bd1_ref, wd2_ref, bd2_ref, cls_ref, dom_ref):
    seq = sums_ref[...] * (1.0 / N)
    sb = seq.reshape(2, 4, F)
    mean = jnp.mean(sb, axis=1)
    d = sb - mean[:, None, :]
    std = jnp.sqrt(jnp.sum(d * d, axis=1) * (1.0 / 3.0))
    feat = jnp.concatenate([mean, std], axis=1)

    def head(w1_ref, b1_ref, w2_ref, b2_ref):
        h = jnp.maximum(
            lax.dot_general(feat, w1_ref[...], (((1,), (0,)), ((), ())),
                            preferred_element_type=jnp.float32)
            + b1_ref[...], 0.0)
        return lax.dot_general(h, w2_ref[...], (((1,), (0,)), ((), ())),
                               preferred_element_type=jnp.float32) + b2_ref[...]

    cls_ref[...] = head(wl1_ref, bl1_ref, wl2_ref, bl2_ref)
    dom_ref[...] = head(wd1_ref, bd1_ref, wd2_ref, bd2_ref)


def _tc_heads(sums, wl1, bl1, wl2, bl2, wd1, bd1, wd2, bd2):
    return pl.pallas_call(
        _head_body,
        out_shape=(jax.ShapeDtypeStruct((2, 2), jnp.float32),
                   jax.ShapeDtypeStruct((2, 2), jnp.float32)),
    )(sums, wl1, bl1.reshape(1, F), wl2, bl2.reshape(1, 2),
      wd1, bd1.reshape(1, F), wd2, bd2.reshape(1, 2))


@jax.jit
def kernel(x, base_adj, edge_index, W1, b1, W2, b2,
           Wl1, bl1, Wl2, bl2, Wd1, bd1, Wd2, bd2):
    src = edge_index[0]
    dst = edge_index[1]
    npad = E_PAD - E
    src_p = jnp.concatenate([src, jnp.zeros((npad,), jnp.int32)])
    dst_p = jnp.concatenate(
        [dst, N + (jnp.arange(npad, dtype=jnp.int32) % TILES)])
    src3 = src_p.reshape(TILES, EPT)
    dst3 = dst_p.reshape(TILES, EPT)
    adjf = base_adj.reshape(N * N)

    h0 = jnp.pad(x.reshape(STEPS, N, F),
                 ((0, 0), (0, NP - N), (0, 0))).reshape(STEPS * NP, F)
    agg1 = _sc_agg(h0, src3, dst3, adjf)
    h1 = _tc_mm_relu(agg1.reshape(STEPS * NP, F), W1, b1)
    agg2 = _sc_agg(h1, src3, dst3, adjf)
    sums = _tc_mm_relu_colsum(agg2[:, :N, :], W2, b2)
    return _tc_heads(sums, Wl1, bl1, Wl2, bl2, Wd1, bd1, Wd2, bd2)


# speedup vs baseline: 1.6901x; 1.6901x over previous
import functools

import jax
import jax.numpy as jnp
from jax import lax
from jax.experimental import pallas as pl
from jax.experimental.pallas import tpu as pltpu
from jax.experimental.pallas import tpu_sc as plsc

N = 10000
E = 320000
F = 128
STEPS = 8
TILES = 16
CHUNK = 128
CPT = 160
EPT = CPT * CHUNK
E_PAD = TILES * EPT
NP = 10752
PIECES = 6
PN = NP // PIECES
PNP = PN + 8
RPT = PN // TILES
SD = 1024


def _sc_agg_body(h_hbm, src_hbm, dst_hbm, adj_hbm, out_hbm,
                 packed, w1, wix, stage_d, gbuf0, gbuf1, sbuf0, sbuf1,
                 ixs0, ixs1, dsts0, dsts1, agg_sh,
                 gsem0, gsem1, ssem0, ssem1):
    core = lax.axis_index("c")
    tid = lax.axis_index("s")
    lane = lax.iota(jnp.int32, 16)
    KG = CHUNK // 16

    def build_ix(ch, g, ixs):
        base = g * NP
        for k in range(KG):
            sl = pl.ds(k * 16, 16)
            ixs[sl] = lax.shift_right_logical(
                packed[pl.ds(ch * CHUNK + k * 16, 16)], 14) + base

    def gstart(buf, sem, ixs):
        pltpu.make_async_copy(h_hbm.at[ixs], buf, sem).start()

    def gwait(buf, sem):
        pltpu.make_async_copy(h_hbm.at[ixs0], buf, sem).wait()

    def sstart(sbuf, sem, dsts):
        pltpu.make_async_copy(
            sbuf, agg_sh.at[dsts.at[0]], sem).start(add=True)

    def swait(sbuf, sem):
        pltpu.make_async_copy(sbuf, agg_sh.at[dsts0.at[0]], sem).wait()

    def process(c, nch, g, gbuf, sbuf, gsem, ssem, ixs, dsts):
        gwait(gbuf, gsem)

        @pl.when(c >= 2)
        def _():
            swait(sbuf, ssem)

        def scale_body(q, carry):
            wv16 = w1[pl.ds(c * CHUNK + q * 16, 16)]
            for l in range(16):
                wb = jnp.full((16,), wv16[l], jnp.float32)
                for f in range(F // 16):
                    sl = pl.ds(f * 16, 16)
                    sbuf[q * 16 + l, sl] = gbuf[q * 16 + l, sl] * wb
            return carry
        lax.fori_loop(0, KG, scale_body, 0)

        nc = c + 2

        @pl.when(nc < nch)
        def _():
            build_ix(nc, g, ixs)
            gstart(gbuf, gsem, ixs)

        for k in range(KG):
            sl = pl.ds(k * 16, 16)
            dsts[0, sl] = packed[pl.ds(c * CHUNK + k * 16, 16)] & 16383
        sstart(sbuf, ssem, dsts)

    def piece_body(qi, pcarry):
        q = core * (PIECES // 2) + qi
        qbase = q * PN

        pltpu.sync_copy(src_hbm.at[tid], packed)

        def pack_blk(blk, carry):
            pltpu.sync_copy(dst_hbm.at[tid].at[pl.ds(blk * SD, SD)], stage_d)

            def pk(k, kc):
                slp = pl.ds(blk * SD + k * 16, 16)
                packed[slp] = packed[slp] * 16384 + stage_d[pl.ds(k * 16, 16)]
                return kc
            lax.fori_loop(0, SD // 16, pk, 0)
            return carry
        lax.fori_loop(0, EPT // SD, pack_blk, 0)

        def wg_body(rr, carry):
            base = rr * 8
            for k in range(8):
                ch = base + k
                for k2 in range(KG):
                    sl = pl.ds(k2 * 16, 16)
                    p = packed[pl.ds(ch * CHUNK + k2 * 16, 16)]
                    wix[k, sl] = (lax.shift_right_logical(p, 14) * N
                                  + (p & 16383))
                pltpu.make_async_copy(
                    adj_hbm.at[wix.at[k]],
                    w1.at[pl.ds(ch * CHUNK, CHUNK)], gsem0).start()
            for k in range(8):
                ch = base + k
                pltpu.make_async_copy(
                    adj_hbm.at[wix.at[k]],
                    w1.at[pl.ds(ch * CHUNK, CHUNK)], gsem0).wait()
            return carry
        lax.fori_loop(0, CPT // 8, wg_body, 0)

        def gat16(v, idx):
            return lax.gather(
                v, idx[:, None],
                lax.GatherDimensionNumbers(
                    offset_dims=(), collapsed_slice_dims=(0,),
                    start_index_map=(0,)),
                slice_sizes=(1,),
                mode=lax.GatherScatterMode.PROMISE_IN_BOUNDS)

        def cpt_body(gr, cnt):
            off = gr * 16
            pv = packed[pl.ds(off, 16)]
            wv = w1[pl.ds(off, 16)]
            dl = (pv & 16383) - qbase
            m = (dl >= 0) & (dl < PN)
            mi = jnp.where(m, 1, 0)
            s = mi
            for d in (1, 2, 4, 8):
                sh = gat16(s, jnp.maximum(lane - d, 0))
                s = s + jnp.where(lane >= d, sh, 0)
            nk = s[15]
            ex = s - mi
            r = jnp.where(m, ex, nk + (lane - ex))
            perm = lane
            for l in range(16):
                perm = jnp.where(lane == r[l], l, perm)
            packed[pl.ds(cnt, 16)] = gat16(pv, perm) - qbase
            w1[pl.ds(cnt, 16)] = gat16(wv, perm)
            return cnt + nk
        cnt = lax.fori_loop(0, EPT // 16, cpt_body, jnp.int32(0))

        c2 = 2 * CHUNK
        pcnt = jnp.maximum((cnt + c2 - 1) // c2 * c2, c2)
        g0 = cnt // 16
        rem = cnt - g0 * 16
        pad_p = PN + (lane & 7)

        @pl.when(cnt < pcnt)
        def _():
            off0 = g0 * 16
            keep = lane < rem
            pv = packed[pl.ds(off0, 16)]
            wv = w1[pl.ds(off0, 16)]
            packed[pl.ds(off0, 16)] = jnp.where(keep, pv, pad_p)
            w1[pl.ds(off0, 16)] = jnp.where(keep, wv, 0.0)

        def pad_body(gg, carry):
            off = gg * 16
            packed[pl.ds(off, 16)] = pad_p
            w1[pl.ds(off, 16)] = jnp.zeros((16,), jnp.float32)
            return carry
        lax.fori_loop(g0 + 1, pcnt // 16, pad_body, 0)

        nch = pcnt // CHUNK

        def step_body(g, carry):
            def zero_body(r, zc):
                for k in range(F // 16):
                    gbuf0[r, pl.ds(k * 16, 16)] = jnp.zeros(
                        (16,), jnp.float32)
                return zc
            lax.fori_loop(0, CHUNK, zero_body, 0)
            for zj in range(RPT // CHUNK):
                pltpu.sync_copy(
                    gbuf0, agg_sh.at[pl.ds(tid * RPT + zj * CHUNK, CHUNK)])
            if RPT % CHUNK:
                pltpu.sync_copy(
                    gbuf0.at[pl.ds(0, RPT % CHUNK)],
                    agg_sh.at[pl.ds(tid * RPT + (RPT // CHUNK) * CHUNK,
                                    RPT % CHUNK)])

            build_ix(0, g, ixs0)
            gstart(gbuf0, gsem0, ixs0)
            build_ix(1, g, ixs1)
            gstart(gbuf1, gsem1, ixs1)

            plsc.subcore_barrier()

            def edge_body(i, ec):
                process(i * 2, nch, g, gbuf0, sbuf0, gsem0, ssem0,
                        ixs0, dsts0)
                process(i * 2 + 1, nch, g, gbuf1, sbuf1, gsem1, ssem1,
                        ixs1, dsts1)
                return ec
            lax.fori_loop(0, nch // 2, edge_body, 0)

            swait(sbuf0, ssem0)
            swait(sbuf1, ssem1)

            plsc.subcore_barrier()

            pltpu.sync_copy(
                agg_sh.at[pl.ds(tid * RPT, RPT)],
                out_hbm.at[g].at[pl.ds(qbase + tid * RPT, RPT)])
            return carry
        lax.fori_loop(0, STEPS, step_body, 0)
        return pcarry
    lax.fori_loop(0, PIECES // 2, piece_body, 0)


def _make_sc_agg():
    return pl.kernel(
        _sc_agg_body,
        out_type=jax.ShapeDtypeStruct((STEPS, NP, F), jnp.float32),
        mesh=plsc.VectorSubcoreMesh(
            core_axis_name="c", subcore_axis_name="s"),
        scratch_types=[
            pltpu.VMEM((EPT,), jnp.int32),
            pltpu.VMEM((EPT,), jnp.float32),
            pltpu.VMEM((8, CHUNK), jnp.int32),
            pltpu.VMEM((SD,), jnp.int32),
            pltpu.VMEM((CHUNK, F), jnp.float32),
            pltpu.VMEM((CHUNK, F), jnp.float32),
            pltpu.VMEM((CHUNK, F), jnp.float32),
            pltpu.VMEM((CHUNK, F), jnp.float32),
            pltpu.VMEM((CHUNK,), jnp.int32),
            pltpu.VMEM((CHUNK,), jnp.int32),
            pltpu.VMEM((1, CHUNK), jnp.int32),
            pltpu.VMEM((1, CHUNK), jnp.int32),
            pltpu.VMEM_SHARED((PNP, F), jnp.float32),
            pltpu.SemaphoreType.DMA,
            pltpu.SemaphoreType.DMA,
            pltpu.SemaphoreType.DMA,
            pltpu.SemaphoreType.DMA,
        ],
    )


_sc_agg = _make_sc_agg()


def _mm_relu_body(a_ref, w_ref, b_ref, o_ref):
    acc = lax.dot_general(a_ref[...], w_ref[...], (((1,), (0,)), ((), ())),
                          preferred_element_type=jnp.float32)
    o_ref[...] = jnp.maximum(acc + b_ref[...], 0.0)


def _tc_mm_relu(a, w, b):
    m = a.shape[0]
    bm = 2048
    return pl.pallas_call(
        _mm_relu_body,
        grid=(m // bm,),
        in_specs=[
            pl.BlockSpec((bm, F), lambda i: (i, 0)),
            pl.BlockSpec((F, F), lambda i: (0, 0)),
            pl.BlockSpec((1, F), lambda i: (0, 0)),
        ],
        out_specs=pl.BlockSpec((bm, F), lambda i: (i, 0)),
        out_shape=jax.ShapeDtypeStruct((m, F), jnp.float32),
    )(a, w, b.reshape(1, F))


def _mm_relu_colsum_body(a_ref, w_ref, b_ref, o_ref):
    j = pl.program_id(1)
    acc = lax.dot_general(a_ref[0], w_ref[...], (((1,), (0,)), ((), ())),
                          preferred_element_type=jnp.float32)
    h = jnp.maximum(acc + b_ref[...], 0.0)
    part = jnp.sum(h, axis=0, keepdims=True)[None]

    @pl.when(j == 0)
    def _():
        o_ref[...] = part

    @pl.when(j > 0)
    def _():
        o_ref[...] = o_ref[...] + part


def _tc_mm_relu_colsum(a, w, b):
    bm = 2000
    bps = N // bm
    return pl.pallas_call(
        _mm_relu_colsum_body,
        grid=(STEPS, bps),
        in_specs=[
            pl.BlockSpec((1, bm, F), lambda s, j: (s, j, 0)),
            pl.BlockSpec((F, F), lambda s, j: (0, 0)),
            pl.BlockSpec((1, F), lambda s, j: (0, 0)),
        ],
        out_specs=pl.BlockSpec((1, 1, F), lambda s, j: (s, 0, 0)),
        out_shape=jax.ShapeDtypeStruct((STEPS, 1, F), jnp.float32),
    )(a, w, b.reshape(1, F)).reshape(STEPS, F)


def _head_body(sums_ref, wl1_ref, bl1_ref, wl2_ref, bl2_ref,
               wd1_ref, bd1_ref, wd2_ref, bd2_ref, cls_ref, dom_ref):
    seq = sums_ref[...] * (1.0 / N)
    sb = seq.reshape(2, 4, F)
    mean = jnp.mean(sb, axis=1)
    d = sb - mean[:, None, :]
    std = jnp.sqrt(jnp.sum(d * d, axis=1) * (1.0 / 3.0))
    feat = jnp.concatenate([mean, std], axis=1)

    def head(w1_ref, b1_ref, w2_ref, b2_ref):
        h = jnp.maximum(
            lax.dot_general(feat, w1_ref[...], (((1,), (0,)), ((), ())),
                            preferred_element_type=jnp.float32)
            + b1_ref[...], 0.0)
        return lax.dot_general(h, w2_ref[...], (((1,), (0,)), ((), ())),
                               preferred_element_type=jnp.float32) + b2_ref[...]

    cls_ref[...] = head(wl1_ref, bl1_ref, wl2_ref, bl2_ref)
    dom_ref[...] = head(wd1_ref, bd1_ref, wd2_ref, bd2_ref)


def _tc_heads(sums, wl1, bl1, wl2, bl2, wd1, bd1, wd2, bd2):
    return pl.pallas_call(
        _head_body,
        out_shape=(jax.ShapeDtypeStruct((2, 2), jnp.float32),
                   jax.ShapeDtypeStruct((2, 2), jnp.float32)),
    )(sums, wl1, bl1.reshape(1, F), wl2, bl2.reshape(1, 2),
      wd1, bd1.reshape(1, F), wd2, bd2.reshape(1, 2))


@jax.jit
def kernel(x, base_adj, edge_index, W1, b1, W2, b2,
           Wl1, bl1, Wl2, bl2, Wd1, bd1, Wd2, bd2):
    src = edge_index[0]
    dst = edge_index[1]
    npad = E_PAD - E
    src_p = jnp.concatenate([src, jnp.zeros((npad,), jnp.int32)])
    dst_p = jnp.concatenate(
        [dst, N + (jnp.arange(npad, dtype=jnp.int32) % TILES)])
    src3 = src_p.reshape(TILES, EPT)
    dst3 = dst_p.reshape(TILES, EPT)
    adjf = base_adj.reshape(N * N)

    h0 = jnp.pad(x.reshape(STEPS, N, F),
                 ((0, 0), (0, NP - N), (0, 0))).reshape(STEPS * NP, F)
    agg1 = _sc_agg(h0, src3, dst3, adjf)
    h1 = _tc_mm_relu(agg1.reshape(STEPS * NP, F), W1, b1)
    agg2 = _sc_agg(h1, src3, dst3, adjf)
    sums = _tc_mm_relu_colsum(agg2[:, :N, :], W2, b2)
    return _tc_heads(sums, Wl1, bl1, Wl2, bl2, Wd1, bd1, Wd2, bd2)
